# Initial kernel scaffold; baseline (speedup 1.0000x reference)
import jax
import jax.numpy as jnp
from jax.experimental import pallas as pl


def kernel(x, W_up1, b_up1, W_blk1, b_blk1, W_up2, b_up2, W_blk2, b_blk2, W_up3, b_up3, W_blk3, b_blk3, prune1_idx, prune2_idx, prune3_idx, src1, dst1, src2, dst2, src3, dst3):
    raise NotImplementedError("write your pallas kernel here")


# speedup vs baseline: 3.1439x; 3.1439x over previous
import functools

import jax
import jax.numpy as jnp
from jax import lax
from jax.experimental import pallas as pl
from jax.experimental.pallas import tpu as pltpu
from jax.experimental.pallas import tpu_sc as plsc

NC, NS, LANES = 2, 16, 16
NW = NC * NS
CH = 128


def _rpad(r):
    q = CH * NW
    return ((r + q - 1) // q) * q


def _nacc(n):
    q = NS * CH
    return ((n + 1 + q - 1) // q) * q


def _tc_matmul(x, w, b2d):
    def body(x_ref, w_ref, b_ref, o_ref):
        o_ref[...] = (
            jnp.dot(x_ref[...], w_ref[...], preferred_element_type=jnp.float32)
            + b_ref[...]
        )
    return pl.pallas_call(
        body,
        out_shape=jax.ShapeDtypeStruct((x.shape[0], w.shape[1]), jnp.float32),
    )(x, w, b2d)


def _tc_combine_matmul(parts, b_in2d, w, b2d):
    def body(p_ref, bi_ref, w_ref, b_ref, o_ref):
        h = jnp.maximum(p_ref[0] + p_ref[1] + bi_ref[...], 0.0)
        o_ref[...] = (
            jnp.dot(h, w_ref[...], preferred_element_type=jnp.float32) + b_ref[...]
        )
    return pl.pallas_call(
        body,
        out_shape=jax.ShapeDtypeStruct((parts.shape[1], w.shape[1]), jnp.float32),
    )(parts, b_in2d, w, b2d)


def _tc_edge_matmul(g, w, e, r_pad):
    kc, cin, cout = w.shape

    def body(g_ref, w_ref, o_ref):
        o_ref[...] = jnp.dot(
            g_ref[...], w_ref[0], preferred_element_type=jnp.float32
        )

    return pl.pallas_call(
        body,
        grid=(kc,),
        in_specs=[
            pl.BlockSpec((e, cin), lambda k: (k, 0)),
            pl.BlockSpec((1, cin, cout), lambda k: (k, 0, 0)),
        ],
        out_specs=pl.BlockSpec((e, cout), lambda k: (k, 0)),
        out_shape=jax.ShapeDtypeStruct((r_pad, cout), jnp.float32),
    )(g, w)


def _tc_final_combine(parts_r, btile):
    def body(p_ref, b_ref, o_ref):
        o_ref[...] = p_ref[0] + p_ref[1] + b_ref[...]
    return pl.pallas_call(
        body,
        out_shape=jax.ShapeDtypeStruct(parts_r.shape[1:], jnp.float32),
    )(parts_r, btile)


def _sc_gather(children, src_pad, prune, n_chunks_w):
    r_pad = src_pad.shape[0]
    c = children.shape[1]
    m = prune.shape[0]
    mesh = plsc.VectorSubcoreMesh(core_axis_name="c", subcore_axis_name="s")

    @functools.partial(
        pl.kernel,
        out_type=jax.ShapeDtypeStruct((r_pad, c), jnp.float32),
        mesh=mesh,
        scratch_types=[
            pltpu.VMEM((m,), jnp.int32),
            pltpu.VMEM((CH,), jnp.int32),
            pltpu.VMEM((CH,), jnp.int32),
            pltpu.VMEM((CH, c), jnp.float32),
            pltpu.SemaphoreType.DMA,
        ],
        compiler_params=pltpu.CompilerParams(needs_layout_passes=False, use_tc_tiling_on_sc=False),
    )
    def kfn(ch_hbm, src_hbm, prune_hbm, g_hbm, prune_v, src_v, cidx_v, rows_v, sem):
        cid = lax.axis_index("c")
        sid = lax.axis_index("s")
        wid = sid * NC + cid
        pltpu.sync_copy(prune_hbm, prune_v)

        def chunk_body(i, carry):
            base = (wid * n_chunks_w + i) * CH
            pltpu.sync_copy(src_hbm.at[pl.ds(base, CH)], src_v)

            def comp(j, carry2):
                idx = src_v[pl.ds(j * LANES, LANES)]
                cidx_v[pl.ds(j * LANES, LANES)] = plsc.load_gather(prune_v, [idx])
                return carry2

            lax.fori_loop(0, CH // LANES, comp, 0)
            pltpu.async_copy(ch_hbm.at[cidx_v], rows_v, sem).wait()
            pltpu.sync_copy(rows_v, g_hbm.at[pl.ds(base, CH)])
            return carry

        lax.fori_loop(0, n_chunks_w, chunk_body, 0)

    return kfn(children, src_pad, prune)


def _sc_scatter(m_rows, dst_pad, n_out, n_chunks_w):
    r_pad, cp = m_rows.shape
    n_acc = _nacc(n_out)
    nz = n_acc // NS
    nzc = nz // CH
    mesh = plsc.VectorSubcoreMesh(core_axis_name="c", subcore_axis_name="s")

    @functools.partial(
        pl.kernel,
        out_type=jax.ShapeDtypeStruct((NC, n_acc, cp), jnp.float32),
        mesh=mesh,
        scratch_types=[
            pltpu.VMEM((CH, cp), jnp.float32),
            pltpu.VMEM((CH,), jnp.int32),
            pltpu.VMEM((CH, cp), jnp.float32),
            pltpu.VMEM_SHARED((n_acc, cp), jnp.float32),
            pltpu.SemaphoreType.DMA,
        ],
        compiler_params=pltpu.CompilerParams(needs_layout_passes=False, use_tc_tiling_on_sc=False),
    )
    def kfn(m_hbm, dst_hbm, out_hbm, rows_v, dst_v, zero_v, acc, sem):
        cid = lax.axis_index("c")
        sid = lax.axis_index("s")
        wid = sid * NC + cid

        def zfill(i, carry):
            def zlane(j, carry2):
                zero_v[i, pl.ds(j * LANES, LANES)] = jnp.zeros(
                    (LANES,), jnp.float32
                )
                return carry2
            lax.fori_loop(0, cp // LANES, zlane, 0)
            return carry

        lax.fori_loop(0, CH, zfill, 0)

        def zacc(i, carry):
            pltpu.sync_copy(zero_v, acc.at[pl.ds(sid * nz + i * CH, CH)])
            return carry

        lax.fori_loop(0, nzc, zacc, 0)
        plsc.subcore_barrier()

        def chunk_body(i, carry):
            base = (wid * n_chunks_w + i) * CH
            pltpu.sync_copy(m_hbm.at[pl.ds(base, CH)], rows_v)
            pltpu.sync_copy(dst_hbm.at[pl.ds(base, CH)], dst_v)
            pltpu.sync_copy(rows_v, acc.at[dst_v], add=True)
            return carry

        lax.fori_loop(0, n_chunks_w, chunk_body, 0)
        plsc.subcore_barrier()

        def wout(i, carry):
            pltpu.sync_copy(
                acc.at[pl.ds(sid * nz + i * CH, CH)],
                out_hbm.at[cid].at[pl.ds(sid * nz + i * CH, CH)],
            )
            return carry

        lax.fori_loop(0, nzc, wout, 0)

    return kfn(m_rows, dst_pad)


def _flatten_pad_edges(src, dst, n_out):
    kc, e = src.shape
    r = kc * e
    r_pad = _rpad(r)
    srcf = jnp.pad(src.reshape(r), (0, r_pad - r))
    dstf = jnp.pad(dst.reshape(r), (0, r_pad - r), constant_values=n_out)
    return srcf, dstf, r, r_pad


def _conv_level(children, prune, src, dst, w_blk, n_out):
    kc, e = src.shape
    srcf, dstf, r, r_pad = _flatten_pad_edges(src, dst, n_out)
    n_chunks_w = r_pad // (CH * NW)
    g = _sc_gather(children, srcf, prune, n_chunks_w)
    m_rows = _tc_edge_matmul(g, w_blk, e, r_pad)
    return _sc_scatter(m_rows, dstf, n_out, n_chunks_w)


def kernel(x, W_up1, b_up1, W_blk1, b_blk1, W_up2, b_up2, W_blk2, b_blk2,
           W_up3, b_up3, W_blk3, b_blk3, prune1_idx, prune2_idx, prune3_idx,
           src1, dst1, src2, dst2, src3, dst3):
    kup = W_up1.shape[0]

    def upw(w):
        return jnp.transpose(w, (1, 0, 2)).reshape(w.shape[1], -1)

    def upb(w, b):
        return jnp.tile(b, (kup,)).reshape(1, -1)

    n2, n1, n0 = prune1_idx.shape[0], prune2_idx.shape[0], prune3_idx.shape[0]

    ch1 = _tc_matmul(x, upw(W_up1), upb(W_up1, b_up1))
    ch1 = ch1.reshape(-1, W_up1.shape[-1])
    p1 = _conv_level(ch1, prune1_idx, src1, dst1, W_blk1, n2)

    ch2 = _tc_combine_matmul(
        p1[:, :n2, :], b_blk1.reshape(1, -1), upw(W_up2), upb(W_up2, b_up2)
    )
    ch2 = ch2.reshape(-1, W_up2.shape[-1])
    p2 = _conv_level(ch2, prune2_idx, src2, dst2, W_blk2, n1)

    ch3 = _tc_combine_matmul(
        p2[:, :n1, :], b_blk2.reshape(1, -1), upw(W_up3), upb(W_up3, b_up3)
    )
    ch3 = ch3.reshape(-1, W_up3.shape[-1])

    cout = W_blk3.shape[-1]
    cpad = LANES
    w3p = jnp.pad(W_blk3, ((0, 0), (0, 0), (0, cpad - cout)))
    p3 = _conv_level(ch3, prune3_idx, src3, dst3, w3p, n0)

    n_acc0 = _nacc(n0)
    parts_r = p3.reshape(NC, n_acc0 * cpad // 128, 128)
    b3tile = jnp.tile(jnp.pad(b_blk3, (0, cpad - cout)), (128 // cpad,)).reshape(1, 128)
    outr = _tc_final_combine(parts_r, b3tile)
    return outr.reshape(n_acc0, cpad)[:n0, :cout]
